# baseline (speedup 1.0000x reference)
import functools

import jax
import jax.numpy as jnp
from jax.experimental import pallas as pl
from jax.experimental.pallas import tpu as pltpu


def _attn_mlp_body(user_ref, item_ref, hist_ref, w1_ref, b1_ref, w2_ref,
                   b2_ref, w3_ref, b3_ref, wo_ref, bo_ref, out_ref):
    user = user_ref[...]
    item = item_ref[...]
    hist = hist_ref[...]

    sim = jnp.sum(hist * item[:, None, :], axis=2)
    m = jnp.max(sim, axis=1, keepdims=True)
    e = jnp.exp(sim - m)
    w = e / jnp.sum(e, axis=1, keepdims=True)
    wh = jnp.sum(hist * w[:, :, None], axis=1)

    x = jnp.concatenate([user, item, wh], axis=1)
    x = jnp.maximum(jnp.dot(x, w1_ref[...],
                            preferred_element_type=jnp.float32) + b1_ref[...], 0.0)
    x = jnp.maximum(jnp.dot(x, w2_ref[...],
                            preferred_element_type=jnp.float32) + b2_ref[...], 0.0)
    x = jnp.maximum(jnp.dot(x, w3_ref[...],
                            preferred_element_type=jnp.float32) + b3_ref[...], 0.0)
    y = jnp.dot(x, wo_ref[...], preferred_element_type=jnp.float32) + bo_ref[...]
    out_ref[...] = jax.nn.sigmoid(y)


def _attn_mlp(user_emb, item_emb, hist_emb, W1, b1, W2, b2, W3, b3, Wo, bo,
              block_b=64):
    B, D = user_emb.shape
    L = hist_emb.shape[1]
    grid = (B // block_b,)
    full = lambda *s: pl.BlockSpec(s, lambda i: (0,) * len(s))
    return pl.pallas_call(
        _attn_mlp_body,
        grid=grid,
        in_specs=[
            pl.BlockSpec((block_b, D), lambda i: (i, 0)),
            pl.BlockSpec((block_b, D), lambda i: (i, 0)),
            pl.BlockSpec((block_b, L, D), lambda i: (i, 0, 0)),
            full(*W1.shape), full(1, b1.shape[0]),
            full(*W2.shape), full(1, b2.shape[0]),
            full(*W3.shape), full(1, b3.shape[0]),
            full(*Wo.shape), full(1, bo.shape[0]),
        ],
        out_specs=pl.BlockSpec((block_b, 1), lambda i: (i, 0)),
        out_shape=jax.ShapeDtypeStruct((B, 1), jnp.float32),
    )(user_emb, item_emb, hist_emb, W1, b1.reshape(1, -1), W2,
      b2.reshape(1, -1), W3, b3.reshape(1, -1), Wo, bo.reshape(1, -1))


@jax.jit
def kernel(user_input, item_input, history_items, user_table, item_table,
           W1, b1, W2, b2, W3, b3, Wo, bo):
    user_emb = jnp.take(user_table, user_input, axis=0)
    item_emb = jnp.take(item_table, item_input, axis=0)
    hist_emb = jnp.take(item_table, history_items.reshape(-1), axis=0)
    hist_emb = hist_emb.reshape(*history_items.shape, item_table.shape[1])
    return _attn_mlp(user_emb, item_emb, hist_emb,
                     W1, b1, W2, b2, W3, b3, Wo, bo)


# speedup vs baseline: 1.0571x; 1.0571x over previous
import functools

import jax
import jax.numpy as jnp
from jax import lax
from jax.experimental import pallas as pl
from jax.experimental.pallas import tpu as pltpu
from jax.experimental.pallas import tpu_sc as plsc


def _sc_gather(user_input, item_input, hist_idx2d, user_table, item_table):
    B = user_input.shape[0]
    CH = hist_idx2d.shape[1]
    BL = hist_idx2d.shape[0] * CH
    D = user_table.shape[1]
    info = plsc.get_sparse_core_info()
    NC, NS = info.num_cores, info.num_subcores
    NW = NC * NS
    b_per_w = B // NW
    h_per_w = BL // NW
    n_ch = h_per_w // CH

    mesh = plsc.VectorSubcoreMesh(core_axis_name="c", subcore_axis_name="s")

    @functools.partial(
        pl.kernel,
        mesh=mesh,
        out_type=(
            jax.ShapeDtypeStruct((B, D), jnp.float32),
            jax.ShapeDtypeStruct((B, D), jnp.float32),
            jax.ShapeDtypeStruct((BL, D), jnp.float32),
        ),
        scratch_types=[
            pltpu.VMEM((b_per_w,), jnp.int32),
            pltpu.VMEM((b_per_w, D), jnp.float32),
            pltpu.VMEM((b_per_w,), jnp.int32),
            pltpu.VMEM((b_per_w, D), jnp.float32),
            pltpu.VMEM((n_ch, CH), jnp.int32),
            pltpu.VMEM((2, CH, D), jnp.float32),
            pltpu.SemaphoreType.DMA,
            pltpu.SemaphoreType.DMA,
            pltpu.SemaphoreType.DMA,
            pltpu.SemaphoreType.DMA,
        ],
        compiler_params=pltpu.CompilerParams(use_tc_tiling_on_sc=False),
    )
    def k(uidx_hbm, iidx_hbm, hidx_hbm, utab_hbm, itab_hbm,
          uout, iout, hout, uidx_v, urows_v, iidx_v, irows_v,
          hidx_v, hrows_v, sem_u, sem_i, sem_h0, sem_h1):
        wid = lax.axis_index("s") * NC + lax.axis_index("c")
        base = wid * b_per_w
        hbase = wid * h_per_w

        pltpu.sync_copy(uidx_hbm.at[pl.ds(base, b_per_w)], uidx_v)
        pltpu.sync_copy(iidx_hbm.at[pl.ds(base, b_per_w)], iidx_v)
        cp_u = pltpu.async_copy(utab_hbm.at[uidx_v], urows_v, sem_u)
        cp_i = pltpu.async_copy(itab_hbm.at[iidx_v], irows_v, sem_i)

        pltpu.sync_copy(hidx_hbm.at[pl.ds(wid * n_ch, n_ch)], hidx_v)

        sems = (sem_h0, sem_h1)

        def start(j, buf):
            return pltpu.async_copy(
                itab_hbm.at[hidx_v.at[j]], hrows_v.at[buf], sems[buf])

        def drain_and_store(j, buf):
            pltpu.make_async_copy(
                itab_hbm.at[hidx_v.at[j]], hrows_v.at[buf],
                sems[buf]).wait()
            pltpu.sync_copy(hrows_v.at[buf],
                            hout.at[pl.ds(hbase + j * CH, CH)])

        start(0, 0)

        def step(t, carry):
            j = 2 * t
            start(j + 1, 1)
            drain_and_store(j, 0)
            start(j + 2, 0)
            drain_and_store(j + 1, 1)
            return carry

        lax.fori_loop(0, (n_ch - 2) // 2, step, 0, unroll=False)
        j = n_ch - 2
        start(j + 1, 1)
        drain_and_store(j, 0)
        drain_and_store(j + 1, 1)

        cp_u.wait()
        pltpu.sync_copy(urows_v, uout.at[pl.ds(base, b_per_w)])
        cp_i.wait()
        pltpu.sync_copy(irows_v, iout.at[pl.ds(base, b_per_w)])

    return k(user_input, item_input, hist_idx2d, user_table, item_table)


def _attn_mlp_body(user_ref, item_ref, hist_ref, w1_ref, b1_ref, w2_ref,
                   b2_ref, w3_ref, b3_ref, wo_ref, bo_ref, out_ref):
    user = user_ref[...]
    item = item_ref[...]
    hist = hist_ref[...]

    sim = jnp.sum(hist * item[:, None, :], axis=2)
    m = jnp.max(sim, axis=1, keepdims=True)
    e = jnp.exp(sim - m)
    w = e / jnp.sum(e, axis=1, keepdims=True)
    wh = jnp.sum(hist * w[:, :, None], axis=1)

    x = jnp.concatenate([user, item, wh], axis=1)
    x = jnp.maximum(jnp.dot(x, w1_ref[...],
                            preferred_element_type=jnp.float32) + b1_ref[...], 0.0)
    x = jnp.maximum(jnp.dot(x, w2_ref[...],
                            preferred_element_type=jnp.float32) + b2_ref[...], 0.0)
    x = jnp.maximum(jnp.dot(x, w3_ref[...],
                            preferred_element_type=jnp.float32) + b3_ref[...], 0.0)
    y = jnp.dot(x, wo_ref[...], preferred_element_type=jnp.float32) + bo_ref[...]
    out_ref[...] = jax.nn.sigmoid(y)


def _attn_mlp(user_emb, item_emb, hist_emb, W1, b1, W2, b2, W3, b3, Wo, bo,
              block_b=64):
    B, D = user_emb.shape
    L = hist_emb.shape[1]
    grid = (B // block_b,)
    full = lambda *s: pl.BlockSpec(s, lambda i: (0,) * len(s))
    return pl.pallas_call(
        _attn_mlp_body,
        grid=grid,
        in_specs=[
            pl.BlockSpec((block_b, D), lambda i: (i, 0)),
            pl.BlockSpec((block_b, D), lambda i: (i, 0)),
            pl.BlockSpec((block_b, L, D), lambda i: (i, 0, 0)),
            full(*W1.shape), full(1, b1.shape[0]),
            full(*W2.shape), full(1, b2.shape[0]),
            full(*W3.shape), full(1, b3.shape[0]),
            full(*Wo.shape), full(1, bo.shape[0]),
        ],
        out_specs=pl.BlockSpec((block_b, 1), lambda i: (i, 0)),
        out_shape=jax.ShapeDtypeStruct((B, 1), jnp.float32),
    )(user_emb, item_emb, hist_emb, W1, b1.reshape(1, -1), W2,
      b2.reshape(1, -1), W3, b3.reshape(1, -1), Wo, bo.reshape(1, -1))


@jax.jit
def kernel(user_input, item_input, history_items, user_table, item_table,
           W1, b1, W2, b2, W3, b3, Wo, bo):
    B, L = history_items.shape
    D = item_table.shape[1]
    hist_idx2d = history_items.reshape(B * L // 128, 128).astype(jnp.int32)
    user_emb, item_emb, hist_emb = _sc_gather(
        user_input.astype(jnp.int32), item_input.astype(jnp.int32),
        hist_idx2d, user_table, item_table)
    hist_emb = hist_emb.reshape(B, L, D)
    return _attn_mlp(user_emb, item_emb, hist_emb,
                     W1, b1, W2, b2, W3, b3, Wo, bo)


# speedup vs baseline: 2.8780x; 2.7226x over previous
import functools

import jax
import jax.numpy as jnp
from jax import lax
from jax.experimental import pallas as pl
from jax.experimental.pallas import tpu as pltpu
from jax.experimental.pallas import tpu_sc as plsc


def _sc_attend(item_input, hist_idx, item_table):
    B = item_input.shape[0]
    HC = hist_idx.shape[1]
    L = 2 * HC
    D = item_table.shape[1]
    info = plsc.get_sparse_core_info()
    NC, NS = info.num_cores, info.num_subcores
    NW = NC * NS
    b_per_w = B // NW

    mesh = plsc.VectorSubcoreMesh(core_axis_name="c", subcore_axis_name="s")

    @functools.partial(
        pl.kernel,
        mesh=mesh,
        out_type=(
            jax.ShapeDtypeStruct((B, D), jnp.float32),
            jax.ShapeDtypeStruct((B, D), jnp.float32),
        ),
        scratch_types=[
            pltpu.VMEM((b_per_w,), jnp.int32),
            pltpu.VMEM((b_per_w, D), jnp.float32),
            pltpu.VMEM((2 * b_per_w, HC), jnp.int32),
            pltpu.VMEM((2, L, D), jnp.float32),
            pltpu.VMEM((b_per_w, D), jnp.float32),
            pltpu.SemaphoreType.DMA,
            pltpu.SemaphoreType.DMA,
            pltpu.SemaphoreType.DMA,
        ],
        compiler_params=pltpu.CompilerParams(use_tc_tiling_on_sc=False,
                                             needs_layout_passes=False),
    )
    def k(iidx_hbm, hidx_hbm, itab_hbm, iout, wout,
          iidx_v, irows_v, hidx_v, hrows_v, wout_v, sem_i, sem_h0, sem_h1):
        wid = lax.axis_index("s") * NC + lax.axis_index("c")
        base = wid * b_per_w

        pltpu.sync_copy(iidx_hbm.at[pl.ds(base, b_per_w)], iidx_v)
        cp_i = pltpu.async_copy(itab_hbm.at[iidx_v], irows_v, sem_i)
        pltpu.sync_copy(hidx_hbm.at[pl.ds(2 * base, 2 * b_per_w)], hidx_v)

        sems = (sem_h0, sem_h1)

        def start(b, buf):
            pltpu.async_copy(itab_hbm.at[hidx_v.at[2 * b]],
                             hrows_v.at[buf, pl.ds(0, HC)], sems[buf])
            pltpu.async_copy(itab_hbm.at[hidx_v.at[2 * b + 1]],
                             hrows_v.at[buf, pl.ds(HC, HC)], sems[buf])

        def drain(b, buf):
            pltpu.make_async_copy(itab_hbm.at[hidx_v.at[2 * b]],
                                  hrows_v.at[buf, pl.ds(0, HC)],
                                  sems[buf]).wait()
            pltpu.make_async_copy(itab_hbm.at[hidx_v.at[2 * b + 1]],
                                  hrows_v.at[buf, pl.ds(HC, HC)],
                                  sems[buf]).wait()

        start(0, 0)
        cp_i.wait()
        pltpu.sync_copy(irows_v, iout.at[pl.ds(base, b_per_w)])

        def compute_row(b, buf):
            q0 = irows_v[b, pl.ds(0, 16)]
            q1 = irows_v[b, pl.ds(16, 16)]
            zero = jnp.zeros((16,), jnp.float32)

            def group(g, carry):
                a0x, a1x, zx, a0y, a1y, zy = carry
                accs = [[a0x, a1x, zx], [a0y, a1y, zy]]
                for kk in range(8):
                    l = g * 8 + kk
                    h0 = hrows_v[buf, l, pl.ds(0, 16)]
                    h1 = hrows_v[buf, l, pl.ds(16, 16)]
                    s = jnp.sum(h0 * q0 + h1 * q1)
                    e = jnp.exp(lax.broadcast_in_dim(s, (16,), ()))
                    a = accs[kk % 2]
                    a[0] = a[0] + e * h0
                    a[1] = a[1] + e * h1
                    a[2] = a[2] + e
                return (accs[0][0], accs[0][1], accs[0][2],
                        accs[1][0], accs[1][1], accs[1][2])

            a0x, a1x, zx, a0y, a1y, zy = lax.fori_loop(
                0, L // 8, group, (zero, zero, zero, zero, zero, zero),
                unroll=False)
            winv = 1.0 / (zx + zy)
            wout_v[b, pl.ds(0, 16)] = (a0x + a0y) * winv
            wout_v[b, pl.ds(16, 16)] = (a1x + a1y) * winv

        def step(t, carry):
            b = 2 * t
            start(b + 1, 1)
            drain(b, 0)
            compute_row(b, 0)
            start(b + 2, 0)
            drain(b + 1, 1)
            compute_row(b + 1, 1)
            return carry

        lax.fori_loop(0, (b_per_w - 2) // 2, step, 0, unroll=False)
        b = b_per_w - 2
        start(b + 1, 1)
        drain(b, 0)
        compute_row(b, 0)
        drain(b + 1, 1)
        compute_row(b + 1, 1)

        pltpu.sync_copy(wout_v, wout.at[pl.ds(base, b_per_w)])

    return k(item_input, hist_idx, item_table)


def _mlp_body(u_ref, i_ref, w_ref, w1_ref, b1_ref, w2_ref, b2_ref,
              w3_ref, b3_ref, wo_ref, bo_ref, out_ref):
    w1 = w1_ref[...]
    x = (jnp.dot(u_ref[...], w1[0:32], preferred_element_type=jnp.float32)
         + jnp.dot(i_ref[...], w1[32:64], preferred_element_type=jnp.float32)
         + jnp.dot(w_ref[...], w1[64:96], preferred_element_type=jnp.float32)
         + b1_ref[...])
    x = jnp.maximum(x, 0.0)
    x = jnp.maximum(jnp.dot(x, w2_ref[...],
                            preferred_element_type=jnp.float32) + b2_ref[...], 0.0)
    x = jnp.maximum(jnp.dot(x, w3_ref[...],
                            preferred_element_type=jnp.float32) + b3_ref[...], 0.0)
    y = jnp.dot(x, wo_ref[...], preferred_element_type=jnp.float32) + bo_ref[...]
    out_ref[...] = jax.nn.sigmoid(y)


def _mlp(user_emb, item_emb, wh, W1, b1, W2, b2, W3, b3, Wo, bo):
    B, D = user_emb.shape
    full = lambda *s: pl.BlockSpec(s, lambda: (0,) * len(s))
    return pl.pallas_call(
        _mlp_body,
        in_specs=[
            full(B, D), full(B, D), full(B, D),
            full(*W1.shape), full(1, b1.shape[0]),
            full(*W2.shape), full(1, b2.shape[0]),
            full(*W3.shape), full(1, b3.shape[0]),
            full(*Wo.shape), full(1, bo.shape[0]),
        ],
        out_specs=full(B, 1),
        out_shape=jax.ShapeDtypeStruct((B, 1), jnp.float32),
    )(user_emb, item_emb, wh, W1, b1.reshape(1, -1), W2,
      b2.reshape(1, -1), W3, b3.reshape(1, -1), Wo, bo.reshape(1, -1))


@jax.jit
def kernel(user_input, item_input, history_items, user_table, item_table,
           W1, b1, W2, b2, W3, b3, Wo, bo):
    B, L = history_items.shape
    hist_idx = history_items.reshape(2 * B, L // 2).astype(jnp.int32)
    user_emb = jnp.take(user_table, user_input, axis=0)
    item_emb, wh = _sc_attend(item_input.astype(jnp.int32), hist_idx,
                              item_table)
    return _mlp(user_emb, item_emb, wh, W1, b1, W2, b2, W3, b3, Wo, bo)
